# Initial kernel scaffold; baseline (speedup 1.0000x reference)
import jax
import jax.numpy as jnp
from jax.experimental import pallas as pl


def kernel(inp_features, inp_positions, out_positions, extents, kernel, bias):
    raise NotImplementedError("write your pallas kernel here")


# speedup vs baseline: 368.5345x; 368.5345x over previous
import functools

import jax
import jax.numpy as jnp
from jax.experimental import pallas as pl
from jax.experimental.pallas import tpu as pltpu

K0, K1, K2 = 3, 3, 3
KPROD = K0 * K1 * K2


def _dense_body(px_ref, py_ref, pz_ref, feats_ref, q_ref, r_ref, wr_ref,
                bias_ref, out_ref):
    px = px_ref[0, :]
    py = py_ref[0, :]
    pz = pz_ref[0, :]
    qx = q_ref[:, 0:1]
    qy = q_ref[:, 1:2]
    qz = q_ref[:, 2:3]
    r = r_ref[:, 0:1]

    dx = px[None, :] - qx
    dy = py[None, :] - qy
    dz = pz[None, :] - qz
    d2 = dx * dx + dy * dy + dz * dz
    mask = (d2 <= r * r).astype(jnp.float32)

    inv_r = 1.0 / r
    relx = dx * inv_r
    rely = dy * inv_r
    relz = dz * inv_r
    norm = jnp.sqrt(jnp.maximum(d2, 1e-20)) * inv_r
    maxabs = jnp.maximum(jnp.maximum(jnp.abs(relx), jnp.abs(rely)),
                         jnp.abs(relz))
    scale = jnp.where(maxabs > 1e-8, norm / jnp.maximum(maxabs, 1e-8), 0.0)

    def axis_w(rel):
        t = jnp.clip(rel * scale + 1.0, 0.0, 2.0)
        lo = jnp.floor(t)
        frac = t - lo
        hi = jnp.minimum(lo + 1.0, 2.0)
        ws = []
        for i in range(3):
            fi = float(i)
            ws.append((lo == fi).astype(jnp.float32) * (1.0 - frac)
                      + (hi == fi).astype(jnp.float32) * frac)
        return ws

    w0 = axis_w(relx)
    w1 = axis_w(rely)
    w2 = axis_w(relz)
    w0 = [w * mask for w in w0]

    feats = feats_ref[...]
    cout = out_ref.shape[1]
    acc = jnp.zeros((q_ref.shape[0], cout), jnp.float32)
    for k in range(KPROD):
        i0, rem = divmod(k, K1 * K2)
        i1, i2 = divmod(rem, K2)
        wk = w0[i0] * w1[i1] * w2[i2]
        wsum_k = jnp.dot(wk, feats,
                         preferred_element_type=jnp.float32)
        acc = acc + jnp.dot(wsum_k, wr_ref[k],
                            preferred_element_type=jnp.float32)

    count = jnp.sum(mask, axis=1, keepdims=True)
    out_ref[...] = acc / jnp.maximum(count, 1.0) + bias_ref[0, :][None, :]


def kernel(inp_features, inp_positions, out_positions, extents, kernel, bias):
    N, cin = inp_features.shape
    M = out_positions.shape[0]
    cout = kernel.shape[-1]
    BM = 128

    px = inp_positions[:, 0].reshape(1, N)
    py = inp_positions[:, 1].reshape(1, N)
    pz = inp_positions[:, 2].reshape(1, N)
    wr = kernel.reshape(KPROD, cin, cout)
    radii = (0.5 * extents).reshape(M, 1)
    bias2 = bias.reshape(1, cout)

    grid = (M // BM,)
    out = pl.pallas_call(
        _dense_body,
        grid=grid,
        in_specs=[
            pl.BlockSpec((1, N), lambda i: (0, 0)),
            pl.BlockSpec((1, N), lambda i: (0, 0)),
            pl.BlockSpec((1, N), lambda i: (0, 0)),
            pl.BlockSpec((N, cin), lambda i: (0, 0)),
            pl.BlockSpec((BM, 3), lambda i: (i, 0)),
            pl.BlockSpec((BM, 1), lambda i: (i, 0)),
            pl.BlockSpec((KPROD, cin, cout), lambda i: (0, 0, 0)),
            pl.BlockSpec((1, cout), lambda i: (0, 0)),
        ],
        out_specs=pl.BlockSpec((BM, cout), lambda i: (i, 0)),
        out_shape=jax.ShapeDtypeStruct((M, cout), jnp.float32),
    )(px, py, pz, inp_features, out_positions, radii, wr, bias2)
    return out
